# baseline (speedup 1.0000x reference)
import jax
import jax.numpy as jnp
from jax.experimental import pallas as pl

_K = 64
_ROWS_PER_BLOCK = 8
_N = 32768
_INT_MIN = -2147483648


def _body(x_ref, o_ref):
    x = x_ref[...]
    R = x.shape[0]
    v = jax.lax.bitcast_convert_type(x, jnp.int32)
    k = jnp.where(v >= 0, v, v ^ jnp.int32(0x7FFFFFFF))

    def cnt_ge(t):
        return jnp.sum((k >= t).astype(jnp.int32), axis=1, keepdims=True)

    c0 = cnt_ge(jnp.zeros((R, 1), jnp.int32))
    t = jnp.where(c0 >= _K, jnp.zeros((R, 1), jnp.int32),
                  jnp.full((R, 1), _INT_MIN, jnp.int32))

    def bit_step(i, t):
        cand = t + (jnp.int32(1) << (jnp.int32(30) - i))
        return jnp.where(cnt_ge(cand) >= _K, cand, t)

    t = jax.lax.fori_loop(0, 31, bit_step, t)
    c_gt = jnp.sum((k > t).astype(jnp.int32), axis=1, keepdims=True)
    m = _K - c_gt
    eq = k == t
    col = jax.lax.broadcasted_iota(jnp.int32, (R, _N), 1)

    def idx_step(i, c):
        cand = c + (jnp.int32(1) << (jnp.int32(14) - i))
        f = jnp.sum((eq & (col <= cand)).astype(jnp.int32), axis=1,
                    keepdims=True)
        return jnp.where(f <= m, cand, c)

    cstar = jax.lax.fori_loop(0, 15, idx_step, jnp.zeros((R, 1), jnp.int32))
    gate = (k > t) | (eq & (col <= cstar))
    o_ref[...] = jnp.where(gate, x, jnp.float32(0.0))


def kernel(x):
    B = x.shape[0]
    return pl.pallas_call(
        _body,
        out_shape=jax.ShapeDtypeStruct(x.shape, x.dtype),
        grid=(B // _ROWS_PER_BLOCK,),
        in_specs=[pl.BlockSpec((_ROWS_PER_BLOCK, _N), lambda i: (i, 0))],
        out_specs=pl.BlockSpec((_ROWS_PER_BLOCK, _N), lambda i: (i, 0)),
    )(x)


# speedup vs baseline: 3.8650x; 3.8650x over previous
import jax
import jax.numpy as jnp
from jax import lax
from jax.experimental import pallas as pl
from jax.experimental.pallas import tpu as pltpu
from jax.experimental.pallas import tpu_sc as plsc

_K = 64
_B = 128
_N = 32768
_NV = _N // 16
_NC = 2
_NS = 16
_ROWS_PER_W = _B // (_NC * _NS)
_CAP = 16384


def _suffix(v):
    return lax.rev(plsc.cumsum(lax.rev(v, (0,))), (0,))


def _extract(vec, i):
    lane = jnp.arange(16, dtype=jnp.int32)
    return jnp.sum(jnp.where(lane == i, vec, 0))


def _level_select(hist_ref, k_rem):
    lane = jnp.arange(16, dtype=jnp.int32)
    tot = jnp.zeros((16,), jnp.int32)
    for r in range(16):
        tot = tot + plsc.load_gather(hist_ref, [lane * 16 + r])
    s = _suffix(tot)
    c0 = jnp.sum((s >= k_rem).astype(jnp.int32)) - 1
    above_chunks = _extract(s, c0) - _extract(tot, c0)
    k2 = k_rem - above_chunks
    bvec = plsc.load_gather(hist_ref, [c0 * 16 + lane])
    sb = _suffix(bvec)
    r0 = jnp.sum((sb >= k2).astype(jnp.int32)) - 1
    sb_r0 = _extract(sb, r0)
    bv_r0 = _extract(bvec, r0)
    d0 = c0 * 16 + r0
    k_next = k2 - (sb_r0 - bv_r0)
    return d0, k_next, bv_r0


def _clear_hist(hist):
    z = jnp.zeros((16,), jnp.int32)
    for r in range(16):
        hist[pl.ds(r * 16, 16)] = z


def _hist_byte(hist, k, shift_hi, shift_d, prefix, extra_mask=None):
    m = lax.shift_right_arithmetic(k, shift_hi) == prefix
    if extra_mask is not None:
        m = m & extra_mask
    idx = lax.shift_right_logical(k, shift_d) & 0xFF
    cnt, last = plsc.scan_count(idx, mask=m)
    plsc.addupdate_scatter(hist, [idx], cnt, mask=last)


def _sc_body(x_hbm, o_hbm, buf, keyb, candb, hist):
    wid = lax.axis_index("s") * _NC + lax.axis_index("c")
    lane = jnp.arange(16, dtype=jnp.int32)

    def do_row(j, carry):
        row = wid * _ROWS_PER_W + j
        pltpu.sync_copy(x_hbm.at[row], buf)
        _clear_hist(hist)

        @plsc.parallel_loop(0, _N, step=16, unroll=16)
        def _p1(i):
            xv = buf[pl.ds(i, 16)]
            v = lax.bitcast_convert_type(xv, jnp.int32)
            k = jnp.where(v >= 0, v, v ^ 0x7FFFFFFF)
            keyb[pl.ds(i, 16)] = k
            idx = lax.shift_right_arithmetic(k, 24) + 128
            cnt, last = plsc.scan_count(idx)
            plsc.addupdate_scatter(hist, [idx], cnt, mask=last)

        d1, k_rem1, n1 = _level_select(hist, jnp.int32(_K))
        prefix1 = d1 - 128

        def compact_path(_):
            zoff = jnp.full((16,), -1, jnp.int32)

            @plsc.parallel_loop(0, _N, step=16, unroll=16, carry=zoff)
            def _p2(i, off):
                k = keyb[pl.ds(i, 16)]
                m = lax.shift_right_arithmetic(k, 24) == prefix1
                mi = m.astype(jnp.int32)
                pos = off + plsc.cumsum(mi)
                plsc.store_scatter(candb, [pos], k, mask=m)
                return off + plsc.all_reduce_population_count(m)

            n1r = lax.shift_left(lax.shift_right_logical(n1 + 15, 4), 4)
            k_rem = k_rem1
            prefix = prefix1
            n_eq = n1
            for shift_hi, shift_d in ((24, 16), (16, 8), (8, 0)):
                _clear_hist(hist)

                @plsc.parallel_loop(0, n1r, step=16)
                def _ml(i, shift_hi=shift_hi, shift_d=shift_d, prefix=prefix):
                    k = candb[pl.ds(i, 16)]
                    valid = (i + lane) < n1
                    _hist_byte(hist, k, shift_hi, shift_d, prefix, valid)

                dl, k_rem, n_eq = _level_select(hist, k_rem)
                prefix = prefix * 256 + dl
            return prefix, k_rem, n_eq

        def full_path(_):
            k_rem = k_rem1
            prefix = prefix1
            n_eq = n1
            for shift_hi, shift_d in ((24, 16), (16, 8), (8, 0)):
                _clear_hist(hist)

                @plsc.parallel_loop(0, _N, step=16, unroll=16)
                def _pm(i, shift_hi=shift_hi, shift_d=shift_d, prefix=prefix):
                    k = keyb[pl.ds(i, 16)]
                    _hist_byte(hist, k, shift_hi, shift_d, prefix)

                dl, k_rem, n_eq = _level_select(hist, k_rem)
                prefix = prefix * 256 + dl
            return prefix, k_rem, n_eq

        t, m_keep, n_eq = lax.cond(n1 <= _CAP, compact_path, full_path, 0)

        @plsc.parallel_loop(0, _N, step=16, unroll=16)
        def _pfin(i):
            k = keyb[pl.ds(i, 16)]
            v = jnp.where(k >= 0, k, k ^ 0x7FFFFFFF)
            xv = lax.bitcast_convert_type(v, jnp.float32)
            buf[pl.ds(i, 16)] = jnp.where(k >= t, xv, 0.0)

        @pl.when(n_eq > m_keep)
        def _fix():
            def fb(i, cnt):
                k = keyb[pl.ds(i * 16, 16)]
                eq = k == t
                eqi = eq.astype(jnp.int32)
                rank = cnt + plsc.cumsum(eqi) - 1
                kill = eq & (rank >= m_keep)
                xv = buf[pl.ds(i * 16, 16)]
                buf[pl.ds(i * 16, 16)] = jnp.where(kill, 0.0, xv)
                return cnt + jnp.sum(eqi)

            lax.fori_loop(0, _NV, fb, jnp.int32(0))

        pltpu.sync_copy(buf, o_hbm.at[row])
        return carry

    lax.fori_loop(0, _ROWS_PER_W, do_row, 0)


def _make(interpret=False):
    mesh = plsc.VectorSubcoreMesh(core_axis_name="c", subcore_axis_name="s")
    return pl.kernel(
        _sc_body,
        out_type=jax.ShapeDtypeStruct((_B, _N), jnp.float32),
        mesh=mesh,
        scratch_types=[
            pltpu.VMEM((_N,), jnp.float32),
            pltpu.VMEM((_N,), jnp.int32),
            pltpu.VMEM((_CAP,), jnp.int32),
            pltpu.VMEM((256,), jnp.int32),
        ],
        compiler_params=pltpu.CompilerParams(needs_layout_passes=False),
        interpret=interpret,
    )


def kernel(x):
    return _make()(x)


# speedup vs baseline: 3.9099x; 1.0116x over previous
import jax
import jax.numpy as jnp
from jax import lax
from jax.experimental import pallas as pl
from jax.experimental.pallas import tpu as pltpu
from jax.experimental.pallas import tpu_sc as plsc

_K = 64
_B = 128
_N = 32768
_NV = _N // 16
_NC = 2
_NS = 16
_ROWS_PER_W = _B // (_NC * _NS)
_CAP = 16384


def _suffix(v):
    return lax.rev(plsc.cumsum(lax.rev(v, (0,))), (0,))


def _popcnt(mask):
    return plsc.all_reduce_population_count(mask)


def _level_select(hist_ref, tmp, k_remv):
    lane = jnp.arange(16, dtype=jnp.int32)
    tot = jnp.zeros((16,), jnp.int32)
    for r in range(16):
        tot = tot + plsc.load_gather(hist_ref, [lane * 16 + r])
    s = _suffix(tot)
    tmp[pl.ds(0, 16)] = s
    tmp[pl.ds(16, 16)] = tot
    c0 = _popcnt(s >= k_remv) - 1
    above_chunks = (plsc.load_gather(tmp, [c0])
                    - plsc.load_gather(tmp, [c0 + 16]))
    k2 = k_remv - above_chunks
    bvec = plsc.load_gather(hist_ref, [c0 * 16 + lane])
    sb = _suffix(bvec)
    tmp[pl.ds(32, 16)] = sb
    tmp[pl.ds(48, 16)] = bvec
    r0 = _popcnt(sb >= k2) - 1
    sb_r0 = plsc.load_gather(tmp, [r0 + 32])
    bv_r0 = plsc.load_gather(tmp, [r0 + 48])
    d0 = c0 * 16 + r0
    k_next = k2 - (sb_r0 - bv_r0)
    return d0, k_next, bv_r0


def _clear_hist(hist):
    z = jnp.zeros((16,), jnp.int32)
    for r in range(16):
        hist[pl.ds(r * 16, 16)] = z


def _hist_byte(hist, k, shift_hi, shift_d, prefixv, extra_mask=None):
    m = lax.shift_right_arithmetic(k, shift_hi) == prefixv
    if extra_mask is not None:
        m = m & extra_mask
    idx = lax.shift_right_logical(k, shift_d) & 0xFF
    cnt, last = plsc.scan_count(idx, mask=m)
    plsc.addupdate_scatter(hist, [idx], cnt, mask=last)


def _sc_body(x_hbm, o_hbm, buf, keyb, candb, hist, tmp):
    wid = lax.axis_index("s") * _NC + lax.axis_index("c")
    lane = jnp.arange(16, dtype=jnp.int32)

    def do_row(j, carry):
        row = wid * _ROWS_PER_W + j
        pltpu.sync_copy(x_hbm.at[row], buf)
        _clear_hist(hist)

        @plsc.parallel_loop(0, _N, step=16, unroll=8)
        def _p1(i):
            xv = buf[pl.ds(i, 16)]
            v = lax.bitcast_convert_type(xv, jnp.int32)
            k = jnp.where(v >= 0, v, v ^ 0x7FFFFFFF)
            keyb[pl.ds(i, 16)] = k
            idx = lax.shift_right_arithmetic(k, 24) + 128
            cnt, last = plsc.scan_count(idx)
            plsc.addupdate_scatter(hist, [idx], cnt, mask=last)

        kinit = jnp.full((16,), _K, jnp.int32)
        d1, k_rem1, n1 = _level_select(hist, tmp, kinit)
        prefix1 = d1 - 128
        n1s = jnp.max(n1)

        def compact_path(_):
            zoff = jnp.full((16,), -1, jnp.int32)

            @plsc.parallel_loop(0, _N, step=16, unroll=8, carry=zoff)
            def _p2(i, off):
                k = keyb[pl.ds(i, 16)]
                m = lax.shift_right_arithmetic(k, 24) == prefix1
                mi = m.astype(jnp.int32)
                pos = off + plsc.cumsum(mi)
                plsc.store_scatter(candb, [pos], k, mask=m)
                return off + _popcnt(m)

            n1r = lax.shift_left(lax.shift_right_logical(n1s + 15, 4), 4)
            k_rem = k_rem1
            prefix = prefix1
            n_eq = n1
            for shift_hi, shift_d in ((24, 16), (16, 8), (8, 0)):
                _clear_hist(hist)

                @plsc.parallel_loop(0, n1r, step=16)
                def _ml(i, shift_hi=shift_hi, shift_d=shift_d, prefix=prefix):
                    k = candb[pl.ds(i, 16)]
                    valid = (i + lane) < n1s
                    _hist_byte(hist, k, shift_hi, shift_d, prefix, valid)

                dl, k_rem, n_eq = _level_select(hist, tmp, k_rem)
                prefix = prefix * 256 + dl
            return prefix, k_rem, n_eq

        def full_path(_):
            k_rem = k_rem1
            prefix = prefix1
            n_eq = n1
            for shift_hi, shift_d in ((24, 16), (16, 8), (8, 0)):
                _clear_hist(hist)

                @plsc.parallel_loop(0, _N, step=16, unroll=8)
                def _pm(i, shift_hi=shift_hi, shift_d=shift_d, prefix=prefix):
                    k = keyb[pl.ds(i, 16)]
                    _hist_byte(hist, k, shift_hi, shift_d, prefix)

                dl, k_rem, n_eq = _level_select(hist, tmp, k_rem)
                prefix = prefix * 256 + dl
            return prefix, k_rem, n_eq

        t, m_keep, n_eq = lax.cond(n1s <= _CAP, compact_path, full_path, 0)

        @plsc.parallel_loop(0, _N, step=16, unroll=8)
        def _pfin(i):
            k = keyb[pl.ds(i, 16)]
            v = jnp.where(k >= 0, k, k ^ 0x7FFFFFFF)
            xv = lax.bitcast_convert_type(v, jnp.float32)
            buf[pl.ds(i, 16)] = jnp.where(k >= t, xv, 0.0)

        @pl.when(jnp.any(n_eq > m_keep))
        def _fix():
            def fb(i, cnt):
                k = keyb[pl.ds(i * 16, 16)]
                eq = k == t
                eqi = eq.astype(jnp.int32)
                rank = cnt + plsc.cumsum(eqi) - 1
                kill = eq & (rank >= m_keep)
                xv = buf[pl.ds(i * 16, 16)]
                buf[pl.ds(i * 16, 16)] = jnp.where(kill, 0.0, xv)
                return cnt + _popcnt(eq)

            lax.fori_loop(0, _NV, fb, jnp.zeros((16,), jnp.int32))

        pltpu.sync_copy(buf, o_hbm.at[row])
        return carry

    lax.fori_loop(0, _ROWS_PER_W, do_row, 0)


def _make(interpret=False):
    mesh = plsc.VectorSubcoreMesh(core_axis_name="c", subcore_axis_name="s")
    return pl.kernel(
        _sc_body,
        out_type=jax.ShapeDtypeStruct((_B, _N), jnp.float32),
        mesh=mesh,
        scratch_types=[
            pltpu.VMEM((_N,), jnp.float32),
            pltpu.VMEM((_N,), jnp.int32),
            pltpu.VMEM((_CAP,), jnp.int32),
            pltpu.VMEM((256,), jnp.int32),
            pltpu.VMEM((64,), jnp.int32),
        ],
        compiler_params=pltpu.CompilerParams(needs_layout_passes=False),
        interpret=interpret,
    )


def kernel(x):
    return _make()(x)


# speedup vs baseline: 4.0889x; 1.0458x over previous
import jax
import jax.numpy as jnp
from jax import lax
from jax.experimental import pallas as pl
from jax.experimental.pallas import tpu as pltpu
from jax.experimental.pallas import tpu_sc as plsc

_K = 64
_B = 128
_N = 32768
_NV = _N // 16
_NC = 2
_NS = 16
_ROWS_PER_W = _B // (_NC * _NS)
_CAP = 16384


def _suffix(v):
    return lax.rev(plsc.cumsum(lax.rev(v, (0,))), (0,))


def _popcnt(mask):
    return plsc.all_reduce_population_count(mask)


def _level_select(hist_ref, tmp, k_remv):
    lane = jnp.arange(16, dtype=jnp.int32)
    tot = jnp.zeros((16,), jnp.int32)
    for r in range(16):
        tot = tot + plsc.load_gather(hist_ref, [lane * 16 + r])
    s = _suffix(tot)
    tmp[pl.ds(0, 16)] = s
    tmp[pl.ds(16, 16)] = tot
    c0 = _popcnt(s >= k_remv) - 1
    above_chunks = (plsc.load_gather(tmp, [c0])
                    - plsc.load_gather(tmp, [c0 + 16]))
    k2 = k_remv - above_chunks
    bvec = plsc.load_gather(hist_ref, [c0 * 16 + lane])
    sb = _suffix(bvec)
    tmp[pl.ds(32, 16)] = sb
    tmp[pl.ds(48, 16)] = bvec
    r0 = _popcnt(sb >= k2) - 1
    sb_r0 = plsc.load_gather(tmp, [r0 + 32])
    bv_r0 = plsc.load_gather(tmp, [r0 + 48])
    d0 = c0 * 16 + r0
    k_next = k2 - (sb_r0 - bv_r0)
    return d0, k_next, bv_r0


def _clear_hist(hist):
    z = jnp.zeros((16,), jnp.int32)
    for r in range(16):
        hist[pl.ds(r * 16, 16)] = z


def _hist_byte(hist, k, shift_hi, shift_d, prefixv, extra_mask=None):
    m = lax.shift_right_arithmetic(k, shift_hi) == prefixv
    if extra_mask is not None:
        m = m & extra_mask
    idx = lax.shift_right_logical(k, shift_d) & 0xFF
    cnt, last = plsc.scan_count(idx, mask=m)
    plsc.addupdate_scatter(hist, [idx], cnt, mask=last)


def _sc_body(x_hbm, o_hbm, buf0, buf1, keyb, candb, hist, tmp, si0, si1):
    wid = lax.axis_index("s") * _NC + lax.axis_index("c")
    lane = jnp.arange(16, dtype=jnp.int32)

    base = wid * _ROWS_PER_W

    def process_row(buf):
        _clear_hist(hist)

        @plsc.parallel_loop(0, _N, step=16, unroll=8)
        def _p1(i):
            xv = buf[pl.ds(i, 16)]
            v = lax.bitcast_convert_type(xv, jnp.int32)
            k = jnp.where(v >= 0, v, v ^ 0x7FFFFFFF)
            keyb[pl.ds(i, 16)] = k
            idx = lax.shift_right_arithmetic(k, 24) + 128
            cnt, last = plsc.scan_count(idx)
            plsc.addupdate_scatter(hist, [idx], cnt, mask=last)

        kinit = jnp.full((16,), _K, jnp.int32)
        d1, k_rem1, n1 = _level_select(hist, tmp, kinit)
        prefix1 = d1 - 128
        n1s = jnp.max(n1)

        def compact_path(_):
            zoff = jnp.full((16,), -1, jnp.int32)

            @plsc.parallel_loop(0, _N, step=16, unroll=8, carry=zoff)
            def _p2(i, off):
                k = keyb[pl.ds(i, 16)]
                m = lax.shift_right_arithmetic(k, 24) == prefix1
                mi = m.astype(jnp.int32)
                pos = off + plsc.cumsum(mi)
                plsc.store_scatter(candb, [pos], k, mask=m)
                return off + _popcnt(m)

            n1r = lax.shift_left(lax.shift_right_logical(n1s + 15, 4), 4)
            k_rem = k_rem1
            prefix = prefix1
            n_eq = n1
            for shift_hi, shift_d in ((24, 16), (16, 8), (8, 0)):
                _clear_hist(hist)

                @plsc.parallel_loop(0, n1r, step=16)
                def _ml(i, shift_hi=shift_hi, shift_d=shift_d, prefix=prefix):
                    k = candb[pl.ds(i, 16)]
                    valid = (i + lane) < n1s
                    _hist_byte(hist, k, shift_hi, shift_d, prefix, valid)

                dl, k_rem, n_eq = _level_select(hist, tmp, k_rem)
                prefix = prefix * 256 + dl
            return prefix, k_rem, n_eq

        def full_path(_):
            k_rem = k_rem1
            prefix = prefix1
            n_eq = n1
            for shift_hi, shift_d in ((24, 16), (16, 8), (8, 0)):
                _clear_hist(hist)

                @plsc.parallel_loop(0, _N, step=16, unroll=8)
                def _pm(i, shift_hi=shift_hi, shift_d=shift_d, prefix=prefix):
                    k = keyb[pl.ds(i, 16)]
                    _hist_byte(hist, k, shift_hi, shift_d, prefix)

                dl, k_rem, n_eq = _level_select(hist, tmp, k_rem)
                prefix = prefix * 256 + dl
            return prefix, k_rem, n_eq

        t, m_keep, n_eq = lax.cond(n1s <= _CAP, compact_path, full_path, 0)

        @plsc.parallel_loop(0, _N, step=16, unroll=8)
        def _pfin(i):
            k = keyb[pl.ds(i, 16)]
            v = jnp.where(k >= 0, k, k ^ 0x7FFFFFFF)
            xv = lax.bitcast_convert_type(v, jnp.float32)
            buf[pl.ds(i, 16)] = jnp.where(k >= t, xv, 0.0)

        @pl.when(jnp.any(n_eq > m_keep))
        def _fix():
            def fb(i, cnt):
                k = keyb[pl.ds(i * 16, 16)]
                eq = k == t
                eqi = eq.astype(jnp.int32)
                rank = cnt + plsc.cumsum(eqi) - 1
                kill = eq & (rank >= m_keep)
                xv = buf[pl.ds(i * 16, 16)]
                buf[pl.ds(i * 16, 16)] = jnp.where(kill, 0.0, xv)
                return cnt + _popcnt(eq)

            lax.fori_loop(0, _NV, fb, jnp.zeros((16,), jnp.int32))

    def cp_in(r, b, sem):
        return pltpu.make_async_copy(x_hbm.at[base + r], b, sem)

    cp_in(0, buf0, si0).start()

    def do_pair(jp, carry):
        r0 = jp * 2
        cp_in(r0 + 1, buf1, si1).start()
        cp_in(r0, buf0, si0).wait()
        process_row(buf0)
        pltpu.sync_copy(buf0, o_hbm.at[base + r0])

        @pl.when(jp < _ROWS_PER_W // 2 - 1)
        def _prefetch():
            cp_in(r0 + 2, buf0, si0).start()

        cp_in(r0 + 1, buf1, si1).wait()
        process_row(buf1)
        pltpu.sync_copy(buf1, o_hbm.at[base + r0 + 1])
        return carry

    lax.fori_loop(0, _ROWS_PER_W // 2, do_pair, 0)


def _make(interpret=False):
    mesh = plsc.VectorSubcoreMesh(core_axis_name="c", subcore_axis_name="s")
    return pl.kernel(
        _sc_body,
        out_type=jax.ShapeDtypeStruct((_B, _N), jnp.float32),
        mesh=mesh,
        scratch_types=[
            pltpu.VMEM((_N,), jnp.float32),
            pltpu.VMEM((_N,), jnp.float32),
            pltpu.VMEM((_N,), jnp.int32),
            pltpu.VMEM((_CAP,), jnp.int32),
            pltpu.VMEM((256,), jnp.int32),
            pltpu.VMEM((64,), jnp.int32),
            pltpu.SemaphoreType.DMA,
            pltpu.SemaphoreType.DMA,
        ],
        compiler_params=pltpu.CompilerParams(needs_layout_passes=False),
        interpret=interpret,
    )


def kernel(x):
    return _make()(x)
